# baseline (speedup 1.0000x reference)
import jax
import jax.numpy as jnp
from jax.experimental import pallas as pl
from jax.experimental.pallas import tpu as pltpu

_CONF_THRES = 0.45
_IOU_THRES = 0.45
_MAX_DET = 300

_B = 4
_N = 20000
_ROWS = 8
_COLS = 2560
_NPAD = _ROWS * _COLS
_NEG_INF = float("-inf")


def _nms_kernel(data_ref, tab_ref, out_ref, d_ref):
    lin = (
        jax.lax.broadcasted_iota(jnp.int32, (_ROWS, _COLS), 0) * _COLS
        + jax.lax.broadcasted_iota(jnp.int32, (_ROWS, _COLS), 1)
    )

    cx = data_ref[:, 0]
    cy = data_ref[:, 1]
    w = data_ref[:, 2]
    h = data_ref[:, 3]
    obj = data_ref[:, 4]
    hw = w * 0.5
    hh = h * 0.5
    x1 = cx - hw
    y1 = cy - hh
    x2 = cx + hw
    y2 = cy + hh
    c0 = data_ref[:, 5] * obj
    c1 = data_ref[:, 6] * obj
    c2 = data_ref[:, 7] * obj
    conf = jnp.maximum(jnp.maximum(c0, c1), c2)
    valid = (obj > _CONF_THRES) & (conf > _CONF_THRES) & (lin[None] < _N)
    d_ref[:, 0] = x1
    d_ref[:, 1] = y1
    d_ref[:, 2] = x2
    d_ref[:, 3] = y2
    d_ref[:, 4] = (x2 - x1) * (y2 - y1)
    d_ref[:, 5] = jnp.where(valid, conf, _NEG_INF)

    def body(i, carry):
        for img in range(_B):
            s = d_ref[img, 5]
            mv = jnp.max(s, keepdims=True)
            okv = mv > _NEG_INF
            cand = jnp.where(s == mv, lin, jnp.int32(2**30))
            idxv = jnp.min(cand, keepdims=True)
            idx = idxv[0, 0]

            t = idx // 64
            rem = idx - t * 64
            su = rem // 8
            u = rem - su * 8
            chunk = tab_ref[img, pl.ds(t, 1)]
            chunk = pltpu.roll(chunk, (8 - su) % 8, axis=1)
            row = pltpu.roll(chunk, (128 - u * 16) % 128, axis=2)[:, 0:1, :]
            cxy = row[:, :, 0:2]
            wh = row[:, :, 2:4]
            hwh = wh * 0.5
            xy1 = cxy - hwh
            xy2 = cxy + hwh
            bobj = row[:, :, 4:5]
            bc0 = row[:, :, 5:6] * bobj
            bc1 = row[:, :, 6:7] * bobj
            bc2 = row[:, :, 7:8] * bobj
            bl = row[:, :, 8:11]
            bx1 = xy1[:, :, 0]
            by1 = xy1[:, :, 1]
            bx2 = xy2[:, :, 0]
            by2 = xy2[:, :, 1]
            barea = (bx2 - bx1) * (by2 - by1)
            bm01 = jnp.maximum(bc0, bc1)
            bconf = jnp.maximum(bm01, bc2)
            bj = jnp.where(bc1 > bc0, 1.0, 0.0)
            bj = jnp.where(bc2 > bm01, 2.0, bj)

            ix1 = jnp.maximum(d_ref[img, 0], bx1)
            iy1 = jnp.maximum(d_ref[img, 1], by1)
            ix2 = jnp.minimum(d_ref[img, 2], bx2)
            iy2 = jnp.minimum(d_ref[img, 3], by2)
            inter = jnp.maximum(ix2 - ix1, 0.0) * jnp.maximum(iy2 - iy1, 0.0)
            iou = inter / (barea + d_ref[img, 4] - inter + 1e-9)
            supp = (iou > _IOU_THRES) | (lin == idxv)
            d_ref[img, 5] = jnp.where(supp, _NEG_INF, s)

            ok3 = okv[:, :, None]
            zf = jnp.float32(0.0)
            out_ref[img, pl.ds(i, 1), :, 0:2] = jnp.where(ok3, xy1, zf)
            out_ref[img, pl.ds(i, 1), :, 2:4] = jnp.where(ok3, xy2, zf)
            out_ref[img, pl.ds(i, 1), :, 4:5] = jnp.where(ok3, bconf, zf)
            out_ref[img, pl.ds(i, 1), :, 5:6] = jnp.where(ok3, bj, zf)
            out_ref[img, pl.ds(i, 1), :, 6:9] = jnp.where(ok3, bl, zf)
        return carry

    jax.lax.fori_loop(0, _MAX_DET, body, 0)


@jax.jit
def kernel(prediction, logits):
    pred_t = prediction.transpose(0, 2, 1)
    pred_t = jnp.pad(pred_t, ((0, 0), (0, 0), (0, _NPAD - _N)))
    data = pred_t.reshape(_B, 8, _ROWS, _COLS)

    raw = jnp.concatenate([prediction, logits], axis=-1)
    raw = jnp.pad(raw, ((0, 0), (0, _NPAD - _N), (0, 5)))
    tab = raw.reshape(_B, _NPAD // 64, 8, 8, 16).reshape(_B, _NPAD // 64, 8, 128)

    out = pl.pallas_call(
        _nms_kernel,
        out_shape=jax.ShapeDtypeStruct((_B, _MAX_DET, 1, 16), jnp.float32),
        scratch_shapes=[pltpu.VMEM((_B, 8, _ROWS, _COLS), jnp.float32)],
    )(data, tab)

    outt = out.reshape(_B, _MAX_DET, 16)
    return outt[:, :, 0:6], outt[:, :, 6:9]


# speedup vs baseline: 2.2302x; 2.2302x over previous
import functools

import jax
import jax.numpy as jnp
from jax import lax
from jax.experimental import pallas as pl
from jax.experimental.pallas import tpu as pltpu
from jax.experimental.pallas import tpu_sc as plsc

_CONF_THRES = 0.45
_IOU_THRES = 0.45
_MAX_DET = 300

_B = 4
_N = 20000
_NPAD = 20480
_NT = 16
_TR = _NPAD // _NT
_TCH = _TR // 16
_CHUNK = 2048
_NCH = _NPAD // _CHUNK
_NEG_INF = float("-inf")
_BIG = 2**30


def _sc_compact(chans_hbm, craw_hbm, ctab_hbm, span_hbm,
                stage_v, craw_v, ctab_v, counts_sh, cntbuf_v, call_v,
                spanbuf_v, sem):
    sid = lax.axis_index("s")
    base = sid * _TR
    lane = lax.iota(jnp.int32, 16)
    _CV = _TR + 16

    def stage_in(img):
        for ch in range(11):
            pltpu.sync_copy(
                chans_hbm.at[pl.ds((img * 11 + ch) * _NPAD + base, _TR)],
                stage_v.at[pl.ds(ch * _TR, _TR)],
            )

    def chunk_valid(j):
        ob = stage_v[pl.ds(4 * _TR + j * 16, 16)]
        s0 = stage_v[pl.ds(5 * _TR + j * 16, 16)] * ob
        s1 = stage_v[pl.ds(6 * _TR + j * 16, 16)] * ob
        s2 = stage_v[pl.ds(7 * _TR + j * 16, 16)] * ob
        conf = jnp.maximum(jnp.maximum(s0, s1), s2)
        return (ob > _CONF_THRES) & (conf > _CONF_THRES)

    cntbuf_v[...] = jnp.zeros((16,), jnp.int32)
    for img in range(_B):
        stage_in(img)

        def cbody(j, cnt):
            v = chunk_valid(j)
            return cnt + jnp.max(jnp.cumsum(jnp.where(v, 1, 0).astype(jnp.int32)))

        cnt = lax.fori_loop(0, _TCH, cbody, jnp.int32(0))
        cnt8 = ((cnt + 7) // 8) * 8
        plsc.store_scatter(
            cntbuf_v,
            [jnp.full((16,), img, jnp.int32)],
            jnp.full((16,), cnt8, jnp.int32),
            mask=lane == img,
        )

    pltpu.sync_copy(cntbuf_v, counts_sh.at[pl.ds(sid * 16, 16)])
    plsc.subcore_barrier()
    pltpu.sync_copy(counts_sh, call_v)

    offs = []
    spans = []
    for img in range(_B):
        row = plsc.load_gather(call_v, [lane * 16 + img])
        cums = jnp.cumsum(row)
        span = jnp.max(cums)
        excl = cums - row
        off = jnp.max(jnp.where(lane == sid, excl, 0))
        offs.append(off)
        spans.append(span)

    spanbuf_v[...] = jnp.zeros((16,), jnp.int32)
    for img in range(_B):
        plsc.store_scatter(
            spanbuf_v,
            [jnp.full((16,), img, jnp.int32)],
            jnp.full((16,), spans[img], jnp.int32),
            mask=lane == img,
        )

    @pl.when(sid == 0)
    def _():
        pltpu.sync_copy(spanbuf_v, span_hbm)

    for img in range(_B):
        stage_in(img)
        off = pl.multiple_of(offs[img], 8)

        def sbody(j, cnt):
            v = chunk_valid(j)
            ranks = jnp.cumsum(jnp.where(v, 1, 0).astype(jnp.int32))
            q = cnt + ranks - 1
            q16 = q * 16
            for ch in range(8):
                vals = stage_v[pl.ds(ch * _TR + j * 16, 16)]
                plsc.store_scatter(craw_v, [ch * _CV + q], vals, mask=v)
                plsc.store_scatter(ctab_v, [q16 + ch], vals, mask=v)
            for ch in range(8, 11):
                vals = stage_v[pl.ds(ch * _TR + j * 16, 16)]
                plsc.store_scatter(ctab_v, [q16 + ch], vals, mask=v)
            return cnt + jnp.max(ranks)

        cnt = lax.fori_loop(0, _TCH, sbody, jnp.int32(0))
        cnt8 = ((cnt + 7) // 8) * 8

        padmask = lane < (cnt8 - cnt)
        padq = cnt + lane
        zeros16 = jnp.zeros((16,), jnp.float32)
        for ch in range(8):
            plsc.store_scatter(craw_v, [ch * _CV + padq], zeros16,
                               mask=padmask)
        for ch in range(16):
            plsc.store_scatter(ctab_v, [padq * 16 + ch], zeros16, mask=padmask)

        npieces = cnt8 // 8

        def pieces(j, do_wait):
            copies = [pltpu.make_async_copy(
                ctab_v.at[pl.ds(j * 128, 128)],
                ctab_hbm.at[pl.ds((img * _NPAD + off + j * 8) * 16, 128)],
                sem,
            )]
            for ch in range(8):
                copies.append(pltpu.make_async_copy(
                    craw_v.at[pl.ds(ch * _CV + j * 8, 8)],
                    craw_hbm.at[
                        pl.ds((img * 8 + ch) * _NPAD + off + j * 8, 8)],
                    sem,
                ))
            for c in copies:
                if do_wait:
                    c.wait()
                else:
                    c.start()

        def issue(j, carry):
            pieces(j, False)
            return carry

        lax.fori_loop(0, npieces, issue, 0)

        def drain(j, carry):
            pieces(j, True)
            return carry

        lax.fori_loop(0, npieces, drain, 0)


_sc_compact_call = functools.partial(
    pl.kernel,
    mesh=plsc.VectorSubcoreMesh(core_axis_name="c", subcore_axis_name="s",
                                num_cores=1),
    out_type=[
        jax.ShapeDtypeStruct((_B * 8 * _NPAD,), jnp.float32),
        jax.ShapeDtypeStruct((_B * _NPAD * 16,), jnp.float32),
        jax.ShapeDtypeStruct((16,), jnp.int32),
    ],
    scratch_types=[
        pltpu.VMEM((11 * _TR,), jnp.float32),
        pltpu.VMEM((8 * (_TR + 16),), jnp.float32),
        pltpu.VMEM(((_TR + 16) * 16,), jnp.float32),
        pltpu.VMEM_SHARED((256,), jnp.int32),
        pltpu.VMEM((16,), jnp.int32),
        pltpu.VMEM((256,), jnp.int32),
        pltpu.VMEM((16,), jnp.int32),
        pltpu.SemaphoreType.DMA,
    ],
    compiler_params=pltpu.CompilerParams(needs_layout_passes=False),
)(_sc_compact)


def _nms_kernel(craw_ref, ctab_ref, span_ref, out_ref, d_ref):
    qio = (
        jax.lax.broadcasted_iota(jnp.int32, (_NCH, 8, 256), 0) * _CHUNK
        + jax.lax.broadcasted_iota(jnp.int32, (_NCH, 8, 256), 1) * 256
        + jax.lax.broadcasted_iota(jnp.int32, (_NCH, 8, 256), 2)
    )
    qio_c = qio[0:1]

    maccs = []
    spans = []
    for img in range(_B):
        cx = craw_ref[img, 0]
        cy = craw_ref[img, 1]
        w = craw_ref[img, 2]
        h = craw_ref[img, 3]
        obj = craw_ref[img, 4]
        hw = w * 0.5
        hh = h * 0.5
        x1 = cx - hw
        y1 = cy - hh
        x2 = cx + hw
        y2 = cy + hh
        c0 = craw_ref[img, 5] * obj
        c1 = craw_ref[img, 6] * obj
        c2 = craw_ref[img, 7] * obj
        conf = jnp.maximum(jnp.maximum(c0, c1), c2)
        span = span_ref[img]
        valid = (obj > _CONF_THRES) & (conf > _CONF_THRES) & (qio < span)
        score = jnp.where(valid, conf, _NEG_INF)
        d_ref[img, 0] = x1
        d_ref[img, 1] = y1
        d_ref[img, 2] = x2
        d_ref[img, 3] = y2
        d_ref[img, 4] = (x2 - x1) * (y2 - y1)
        d_ref[img, 5] = score
        maccs.append(jnp.max(score, axis=0))
        spans.append(span)

    ncmax = (jnp.maximum(
        jnp.maximum(spans[0], spans[1]),
        jnp.maximum(spans[2], spans[3])) + (_CHUNK - 1)) // _CHUNK

    def body(i, maccs):
        ms = [jnp.max(maccs[img]) for img in range(_B)]
        idxs = []
        for img in range(_B):
            cand = jnp.where(d_ref[img, 5] == ms[img], qio, _BIG)
            idxs.append(jnp.min(cand))

        rows = []
        for img in range(_B):
            idx = idxs[img]
            t = idx // 64
            rem = idx - t * 64
            su = rem // 8
            u = rem - su * 8
            chunk = ctab_ref[img, pl.ds(t, 1)]
            chunk = pltpu.roll(chunk, (8 - su) % 8, axis=1)
            rows.append(pltpu.roll(chunk, (128 - u * 16) % 128, axis=2))

        binfo = []
        for img in range(_B):
            chunk = rows[img]
            bcx = chunk[0, 0, 0]
            bcy = chunk[0, 0, 1]
            bw = chunk[0, 0, 2]
            bh = chunk[0, 0, 3]
            bobj = chunk[0, 0, 4]
            bc0 = chunk[0, 0, 5] * bobj
            bc1 = chunk[0, 0, 6] * bobj
            bc2 = chunk[0, 0, 7] * bobj
            bl0 = chunk[0, 0, 8]
            bl1 = chunk[0, 0, 9]
            bl2 = chunk[0, 0, 10]
            bhw = bw * 0.5
            bhh = bh * 0.5
            bx1 = bcx - bhw
            by1 = bcy - bhh
            bx2 = bcx + bhw
            by2 = bcy + bhh
            barea = (bx2 - bx1) * (by2 - by1)
            bm01 = jnp.maximum(bc0, bc1)
            bconf = jnp.maximum(bm01, bc2)
            bj = jnp.where(bc1 > bc0, 1.0, 0.0)
            bj = jnp.where(bc2 > bm01, 2.0, bj)
            binfo.append((bx1, by1, bx2, by2, barea, bconf, bj, bl0, bl1, bl2))

        def supp_body(jc, accs):
            new = []
            for img in range(_B):
                bx1, by1, bx2, by2, barea = binfo[img][:5]
                sc = d_ref[img, 5, pl.ds(jc, 1)][0]
                ix1 = jnp.maximum(d_ref[img, 0, pl.ds(jc, 1)][0], bx1)
                iy1 = jnp.maximum(d_ref[img, 1, pl.ds(jc, 1)][0], by1)
                ix2 = jnp.minimum(d_ref[img, 2, pl.ds(jc, 1)][0], bx2)
                iy2 = jnp.minimum(d_ref[img, 3, pl.ds(jc, 1)][0], by2)
                inter = (jnp.maximum(ix2 - ix1, 0.0)
                         * jnp.maximum(iy2 - iy1, 0.0))
                iou = inter / (barea + d_ref[img, 4, pl.ds(jc, 1)][0]
                               - inter + 1e-9)
                qc = qio_c[0] + jc * _CHUNK
                supp = (iou > _IOU_THRES) | (qc == idxs[img])
                snew = jnp.where(supp, _NEG_INF, sc)
                d_ref[img, 5, pl.ds(jc, 1)] = snew[None]
                new.append(jnp.maximum(accs[img], snew))
            return tuple(new)

        init = tuple(jnp.full((8, 256), _NEG_INF, jnp.float32)
                     for _ in range(_B))
        maccs = lax.fori_loop(0, ncmax, supp_body, init)

        for img in range(_B):
            bx1, by1, bx2, by2, barea, bconf, bj, bl0, bl1, bl2 = binfo[img]
            mok = jnp.where(ms[img] > _NEG_INF, 1.0, 0.0)
            zero = jnp.float32(0.0)
            vals = jnp.stack(
                [bx1, by1, bx2, by2, bconf, bj, bl0, bl1, bl2,
                 zero, zero, zero, zero, zero, zero, zero]
            ) * mok
            out_ref[img, pl.ds(i, 1)] = vals.reshape(1, 1, 16)
        return maccs

    lax.fori_loop(0, _MAX_DET, body, tuple(maccs))


@jax.jit
def kernel(prediction, logits):
    allc = jnp.concatenate([prediction, logits], axis=-1)
    chans = jnp.pad(allc.transpose(0, 2, 1), ((0, 0), (0, 0), (0, _NPAD - _N)))

    craw, ctab, span = _sc_compact_call(chans.reshape(-1))

    out = pl.pallas_call(
        _nms_kernel,
        in_specs=[
            pl.BlockSpec(memory_space=pltpu.VMEM),
            pl.BlockSpec(memory_space=pltpu.VMEM),
            pl.BlockSpec(memory_space=pltpu.SMEM),
        ],
        out_specs=pl.BlockSpec(memory_space=pltpu.VMEM),
        scratch_shapes=[pltpu.VMEM((_B, 6, _NCH, 8, 256), jnp.float32)],
        out_shape=jax.ShapeDtypeStruct((_B, _MAX_DET, 1, 16), jnp.float32),
    )(
        craw.reshape(_B, 8, _NCH, 8, 256),
        ctab.reshape(_B, _NPAD // 64, 8, 128),
        span,
    )

    outt = out.reshape(_B, _MAX_DET, 16)
    return outt[:, :, 0:6], outt[:, :, 6:9]


# speedup vs baseline: 2.6134x; 1.1719x over previous
import functools

import jax
import jax.numpy as jnp
from jax import lax
from jax.experimental import pallas as pl
from jax.experimental.pallas import tpu as pltpu
from jax.experimental.pallas import tpu_sc as plsc

_CONF_THRES = 0.45
_IOU_THRES = 0.45
_MAX_DET = 300

_B = 4
_N = 20000
_NPAD = 20480
_NT = 16
_TR = _NPAD // _NT
_TCH = _TR // 16
_CHUNK = 2048
_NCH = _NPAD // _CHUNK
_NEG_INF = float("-inf")
_BIG = 2**30


def _sc_compact(chans_hbm, craw_hbm, ctab_hbm, span_hbm,
                stage_v, craw_v, ctab_v, counts_sh, cntbuf_v, call_v,
                spanbuf_v, sem):
    sid = lax.axis_index("s")
    base = sid * _TR
    lane = lax.iota(jnp.int32, 16)
    _CV = _TR + 16

    def stage_in(img, chs):
        copies = [
            pltpu.make_async_copy(
                chans_hbm.at[pl.ds((img * 11 + ch) * _NPAD + base, _TR)],
                stage_v.at[pl.ds(ch * _TR, _TR)],
                sem,
            )
            for ch in chs
        ]
        for c in copies:
            c.start()
        for c in copies:
            c.wait()

    def chunk_valid(j):
        ob = stage_v[pl.ds(4 * _TR + j * 16, 16)]
        s0 = stage_v[pl.ds(5 * _TR + j * 16, 16)] * ob
        s1 = stage_v[pl.ds(6 * _TR + j * 16, 16)] * ob
        s2 = stage_v[pl.ds(7 * _TR + j * 16, 16)] * ob
        conf = jnp.maximum(jnp.maximum(s0, s1), s2)
        return (ob > _CONF_THRES) & (conf > _CONF_THRES)

    cntbuf_v[...] = jnp.zeros((16,), jnp.int32)
    for img in range(_B):
        stage_in(img, range(4, 8))

        def cbody(j, cnt):
            v = chunk_valid(j)
            return cnt + jnp.max(jnp.cumsum(jnp.where(v, 1, 0).astype(jnp.int32)))

        cnt = lax.fori_loop(0, _TCH, cbody, jnp.int32(0))
        cnt8 = ((cnt + 7) // 8) * 8
        plsc.store_scatter(
            cntbuf_v,
            [jnp.full((16,), img, jnp.int32)],
            jnp.full((16,), cnt8, jnp.int32),
            mask=lane == img,
        )

    pltpu.sync_copy(cntbuf_v, counts_sh.at[pl.ds(sid * 16, 16)])
    plsc.subcore_barrier()
    pltpu.sync_copy(counts_sh, call_v)

    offs = []
    spans = []
    for img in range(_B):
        row = plsc.load_gather(call_v, [lane * 16 + img])
        cums = jnp.cumsum(row)
        span = jnp.max(cums)
        excl = cums - row
        off = jnp.max(jnp.where(lane == sid, excl, 0))
        offs.append(off)
        spans.append(span)

    spanbuf_v[...] = jnp.zeros((16,), jnp.int32)
    for img in range(_B):
        plsc.store_scatter(
            spanbuf_v,
            [jnp.full((16,), img, jnp.int32)],
            jnp.full((16,), spans[img], jnp.int32),
            mask=lane == img,
        )

    @pl.when(sid == 0)
    def _():
        pltpu.sync_copy(spanbuf_v, span_hbm)

    for img in range(_B):
        stage_in(img, range(11))
        off = pl.multiple_of(offs[img], 8)

        def sbody(j, cnt):
            v = chunk_valid(j)
            ranks = jnp.cumsum(jnp.where(v, 1, 0).astype(jnp.int32))
            q = cnt + ranks - 1
            q16 = q * 16
            for ch in range(8):
                vals = stage_v[pl.ds(ch * _TR + j * 16, 16)]
                plsc.store_scatter(craw_v, [ch * _CV + q], vals, mask=v)
                plsc.store_scatter(ctab_v, [q16 + ch], vals, mask=v)
            for ch in range(8, 11):
                vals = stage_v[pl.ds(ch * _TR + j * 16, 16)]
                plsc.store_scatter(ctab_v, [q16 + ch], vals, mask=v)
            return cnt + jnp.max(ranks)

        cnt = lax.fori_loop(0, _TCH, sbody, jnp.int32(0))
        cnt8 = ((cnt + 7) // 8) * 8

        padmask = lane < (cnt8 - cnt)
        padq = cnt + lane
        zeros16 = jnp.zeros((16,), jnp.float32)
        for ch in range(8):
            plsc.store_scatter(craw_v, [ch * _CV + padq], zeros16,
                               mask=padmask)
        for ch in range(16):
            plsc.store_scatter(ctab_v, [padq * 16 + ch], zeros16, mask=padmask)

        n64 = cnt8 // 64
        tbase = pl.multiple_of(n64 * 64, 8)
        ntail = (cnt8 - n64 * 64) // 8

        def pieces(j, sz, pbase, do_wait):
            p = pbase + j * sz
            copies = [pltpu.make_async_copy(
                ctab_v.at[pl.ds(p * 16, sz * 16)],
                ctab_hbm.at[pl.ds((img * _NPAD + off + p) * 16, sz * 16)],
                sem,
            )]
            for ch in range(8):
                copies.append(pltpu.make_async_copy(
                    craw_v.at[pl.ds(ch * _CV + p, sz)],
                    craw_hbm.at[pl.ds((img * 8 + ch) * _NPAD + off + p, sz)],
                    sem,
                ))
            for c in copies:
                if do_wait:
                    c.wait()
                else:
                    c.start()

        lax.fori_loop(0, n64, lambda j, c: (pieces(j, 64, 0, False), c)[1], 0)
        lax.fori_loop(0, ntail,
                      lambda j, c: (pieces(j, 8, tbase, False), c)[1], 0)
        lax.fori_loop(0, n64, lambda j, c: (pieces(j, 64, 0, True), c)[1], 0)
        lax.fori_loop(0, ntail,
                      lambda j, c: (pieces(j, 8, tbase, True), c)[1], 0)


_sc_compact_call = functools.partial(
    pl.kernel,
    mesh=plsc.VectorSubcoreMesh(core_axis_name="c", subcore_axis_name="s",
                                num_cores=1),
    out_type=[
        jax.ShapeDtypeStruct((_B * 8 * _NPAD,), jnp.float32),
        jax.ShapeDtypeStruct((_B * _NPAD * 16,), jnp.float32),
        jax.ShapeDtypeStruct((16,), jnp.int32),
    ],
    scratch_types=[
        pltpu.VMEM((11 * _TR,), jnp.float32),
        pltpu.VMEM((8 * (_TR + 16),), jnp.float32),
        pltpu.VMEM(((_TR + 16) * 16,), jnp.float32),
        pltpu.VMEM_SHARED((256,), jnp.int32),
        pltpu.VMEM((16,), jnp.int32),
        pltpu.VMEM((256,), jnp.int32),
        pltpu.VMEM((16,), jnp.int32),
        pltpu.SemaphoreType.DMA,
    ],
    compiler_params=pltpu.CompilerParams(needs_layout_passes=False),
)(_sc_compact)


def _nms_kernel(craw_ref, ctab_ref, span_ref, out_ref, d_ref):
    qio = (
        jax.lax.broadcasted_iota(jnp.int32, (_NCH, 8, 256), 0) * _CHUNK
        + jax.lax.broadcasted_iota(jnp.int32, (_NCH, 8, 256), 1) * 256
        + jax.lax.broadcasted_iota(jnp.int32, (_NCH, 8, 256), 2)
    )
    qio_c = qio[0:1]

    maccs = []
    spans = []
    for img in range(_B):
        cx = craw_ref[img, 0]
        cy = craw_ref[img, 1]
        w = craw_ref[img, 2]
        h = craw_ref[img, 3]
        obj = craw_ref[img, 4]
        hw = w * 0.5
        hh = h * 0.5
        x1 = cx - hw
        y1 = cy - hh
        x2 = cx + hw
        y2 = cy + hh
        c0 = craw_ref[img, 5] * obj
        c1 = craw_ref[img, 6] * obj
        c2 = craw_ref[img, 7] * obj
        conf = jnp.maximum(jnp.maximum(c0, c1), c2)
        span = span_ref[img]
        valid = (obj > _CONF_THRES) & (conf > _CONF_THRES) & (qio < span)
        score = jnp.where(valid, conf, _NEG_INF)
        d_ref[img, 0] = x1
        d_ref[img, 1] = y1
        d_ref[img, 2] = x2
        d_ref[img, 3] = y2
        d_ref[img, 4] = (x2 - x1) * (y2 - y1)
        d_ref[img, 5] = score
        maccs.append(jnp.max(score, axis=0))
        spans.append(span)

    ncmax = (jnp.maximum(
        jnp.maximum(spans[0], spans[1]),
        jnp.maximum(spans[2], spans[3])) + (_CHUNK - 1)) // _CHUNK

    def body(i, maccs):
        ms = [jnp.max(maccs[img]) for img in range(_B)]
        idxs = []
        for img in range(_B):
            cand = jnp.where(d_ref[img, 5] == ms[img], qio, _BIG)
            idxs.append(jnp.min(cand))

        rows = []
        for img in range(_B):
            idx = idxs[img]
            t = idx // 64
            rem = idx - t * 64
            su = rem // 8
            u = rem - su * 8
            chunk = ctab_ref[img, pl.ds(t, 1)]
            chunk = pltpu.roll(chunk, (8 - su) % 8, axis=1)
            rows.append(pltpu.roll(chunk, (128 - u * 16) % 128, axis=2))

        binfo = []
        for img in range(_B):
            chunk = rows[img]
            bcx = chunk[0, 0, 0]
            bcy = chunk[0, 0, 1]
            bw = chunk[0, 0, 2]
            bh = chunk[0, 0, 3]
            bobj = chunk[0, 0, 4]
            bc0 = chunk[0, 0, 5] * bobj
            bc1 = chunk[0, 0, 6] * bobj
            bc2 = chunk[0, 0, 7] * bobj
            bl0 = chunk[0, 0, 8]
            bl1 = chunk[0, 0, 9]
            bl2 = chunk[0, 0, 10]
            bhw = bw * 0.5
            bhh = bh * 0.5
            bx1 = bcx - bhw
            by1 = bcy - bhh
            bx2 = bcx + bhw
            by2 = bcy + bhh
            barea = (bx2 - bx1) * (by2 - by1)
            bm01 = jnp.maximum(bc0, bc1)
            bconf = jnp.maximum(bm01, bc2)
            bj = jnp.where(bc1 > bc0, 1.0, 0.0)
            bj = jnp.where(bc2 > bm01, 2.0, bj)
            binfo.append((bx1, by1, bx2, by2, barea, bconf, bj, bl0, bl1, bl2))

        def supp_body(jc, accs):
            new = []
            for img in range(_B):
                bx1, by1, bx2, by2, barea = binfo[img][:5]
                sc = d_ref[img, 5, pl.ds(jc, 1)][0]
                ix1 = jnp.maximum(d_ref[img, 0, pl.ds(jc, 1)][0], bx1)
                iy1 = jnp.maximum(d_ref[img, 1, pl.ds(jc, 1)][0], by1)
                ix2 = jnp.minimum(d_ref[img, 2, pl.ds(jc, 1)][0], bx2)
                iy2 = jnp.minimum(d_ref[img, 3, pl.ds(jc, 1)][0], by2)
                inter = (jnp.maximum(ix2 - ix1, 0.0)
                         * jnp.maximum(iy2 - iy1, 0.0))
                iou = inter / (barea + d_ref[img, 4, pl.ds(jc, 1)][0]
                               - inter + 1e-9)
                qc = qio_c[0] + jc * _CHUNK
                supp = (iou > _IOU_THRES) | (qc == idxs[img])
                snew = jnp.where(supp, _NEG_INF, sc)
                d_ref[img, 5, pl.ds(jc, 1)] = snew[None]
                new.append(jnp.maximum(accs[img], snew))
            return tuple(new)

        init = tuple(jnp.full((8, 256), _NEG_INF, jnp.float32)
                     for _ in range(_B))
        maccs = lax.fori_loop(0, ncmax, supp_body, init)

        for img in range(_B):
            bx1, by1, bx2, by2, barea, bconf, bj, bl0, bl1, bl2 = binfo[img]
            mok = jnp.where(ms[img] > _NEG_INF, 1.0, 0.0)
            zero = jnp.float32(0.0)
            vals = jnp.stack(
                [bx1, by1, bx2, by2, bconf, bj, bl0, bl1, bl2,
                 zero, zero, zero, zero, zero, zero, zero]
            ) * mok
            out_ref[img, pl.ds(i, 1)] = vals.reshape(1, 1, 16)
        return maccs

    lax.fori_loop(0, _MAX_DET, body, tuple(maccs))


@jax.jit
def kernel(prediction, logits):
    allc = jnp.concatenate([prediction, logits], axis=-1)
    chans = jnp.pad(allc.transpose(0, 2, 1), ((0, 0), (0, 0), (0, _NPAD - _N)))

    craw, ctab, span = _sc_compact_call(chans.reshape(-1))

    out = pl.pallas_call(
        _nms_kernel,
        in_specs=[
            pl.BlockSpec(memory_space=pltpu.VMEM),
            pl.BlockSpec(memory_space=pltpu.VMEM),
            pl.BlockSpec(memory_space=pltpu.SMEM),
        ],
        out_specs=pl.BlockSpec(memory_space=pltpu.VMEM),
        scratch_shapes=[pltpu.VMEM((_B, 6, _NCH, 8, 256), jnp.float32)],
        out_shape=jax.ShapeDtypeStruct((_B, _MAX_DET, 1, 16), jnp.float32),
    )(
        craw.reshape(_B, 8, _NCH, 8, 256),
        ctab.reshape(_B, _NPAD // 64, 8, 128),
        span,
    )

    outt = out.reshape(_B, _MAX_DET, 16)
    return outt[:, :, 0:6], outt[:, :, 6:9]


# speedup vs baseline: 2.6949x; 1.0312x over previous
import functools

import jax
import jax.numpy as jnp
from jax import lax
from jax.experimental import pallas as pl
from jax.experimental.pallas import tpu as pltpu
from jax.experimental.pallas import tpu_sc as plsc

_CONF_THRES = 0.45
_IOU_THRES = 0.45
_MAX_DET = 300

_B = 4
_N = 20000
_NPAD = 20480
_NT = 16
_TR = _NPAD // _NT
_TCH = _TR // 16
_CHUNK = 2048
_NCH = _NPAD // _CHUNK
_NEG_INF = float("-inf")
_BIG = 2**30


def _sc_compact(chans_hbm, craw_hbm, ctab_hbm, span_hbm,
                stage_v, craw_v, ctab_v, counts_sh, cntbuf_v, call_v,
                spanbuf_v, sem):
    sid = lax.axis_index("s")
    base = sid * _TR
    lane = lax.iota(jnp.int32, 16)
    _CV = _TR + 16

    def stage_in(img, chs):
        copies = [
            pltpu.make_async_copy(
                chans_hbm.at[pl.ds((img * 11 + ch) * _NPAD + base, _TR)],
                stage_v.at[pl.ds(ch * _TR, _TR)],
                sem,
            )
            for ch in chs
        ]
        for c in copies:
            c.start()
        for c in copies:
            c.wait()

    def chunk_valid(j):
        ob = stage_v[pl.ds(4 * _TR + j * 16, 16)]
        s0 = stage_v[pl.ds(5 * _TR + j * 16, 16)] * ob
        s1 = stage_v[pl.ds(6 * _TR + j * 16, 16)] * ob
        s2 = stage_v[pl.ds(7 * _TR + j * 16, 16)] * ob
        conf = jnp.maximum(jnp.maximum(s0, s1), s2)
        return (ob > _CONF_THRES) & (conf > _CONF_THRES)

    cntbuf_v[...] = jnp.zeros((16,), jnp.int32)
    for img in range(_B):
        stage_in(img, range(4, 8))

        def cbody(j, cnt):
            v = chunk_valid(j)
            return cnt + jnp.max(jnp.cumsum(jnp.where(v, 1, 0).astype(jnp.int32)))

        cnt = lax.fori_loop(0, _TCH, cbody, jnp.int32(0))
        cnt8 = ((cnt + 7) // 8) * 8
        plsc.store_scatter(
            cntbuf_v,
            [jnp.full((16,), img, jnp.int32)],
            jnp.full((16,), cnt8, jnp.int32),
            mask=lane == img,
        )

    pltpu.sync_copy(cntbuf_v, counts_sh.at[pl.ds(sid * 16, 16)])
    plsc.subcore_barrier()
    pltpu.sync_copy(counts_sh, call_v)

    offs = []
    spans = []
    for img in range(_B):
        row = plsc.load_gather(call_v, [lane * 16 + img])
        cums = jnp.cumsum(row)
        span = jnp.max(cums)
        excl = cums - row
        off = jnp.max(jnp.where(lane == sid, excl, 0))
        offs.append(off)
        spans.append(span)

    spanbuf_v[...] = jnp.zeros((16,), jnp.int32)
    for img in range(_B):
        plsc.store_scatter(
            spanbuf_v,
            [jnp.full((16,), img, jnp.int32)],
            jnp.full((16,), spans[img], jnp.int32),
            mask=lane == img,
        )

    @pl.when(sid == 0)
    def _():
        pltpu.sync_copy(spanbuf_v, span_hbm)

    for img in range(_B):
        stage_in(img, range(11))
        off = pl.multiple_of(offs[img], 8)

        def sbody(j, cnt):
            v = chunk_valid(j)
            ranks = jnp.cumsum(jnp.where(v, 1, 0).astype(jnp.int32))
            q = cnt + ranks - 1
            q16 = q * 16
            for ch in range(8):
                vals = stage_v[pl.ds(ch * _TR + j * 16, 16)]
                plsc.store_scatter(craw_v, [ch * _CV + q], vals, mask=v)
                plsc.store_scatter(ctab_v, [q16 + ch], vals, mask=v)
            for ch in range(8, 11):
                vals = stage_v[pl.ds(ch * _TR + j * 16, 16)]
                plsc.store_scatter(ctab_v, [q16 + ch], vals, mask=v)
            return cnt + jnp.max(ranks)

        cnt = lax.fori_loop(0, _TCH, sbody, jnp.int32(0))
        cnt8 = ((cnt + 7) // 8) * 8

        padmask = lane < (cnt8 - cnt)
        padq = cnt + lane
        zeros16 = jnp.zeros((16,), jnp.float32)
        for ch in range(8):
            plsc.store_scatter(craw_v, [ch * _CV + padq], zeros16,
                               mask=padmask)
        for ch in range(16):
            plsc.store_scatter(ctab_v, [padq * 16 + ch], zeros16, mask=padmask)

        n64 = cnt8 // 64
        tbase = pl.multiple_of(n64 * 64, 8)
        ntail = (cnt8 - n64 * 64) // 8

        def pieces(j, sz, pbase, do_wait):
            p = pbase + j * sz
            copies = [pltpu.make_async_copy(
                ctab_v.at[pl.ds(p * 16, sz * 16)],
                ctab_hbm.at[pl.ds((img * _NPAD + off + p) * 16, sz * 16)],
                sem,
            )]
            for ch in range(8):
                copies.append(pltpu.make_async_copy(
                    craw_v.at[pl.ds(ch * _CV + p, sz)],
                    craw_hbm.at[pl.ds((img * 8 + ch) * _NPAD + off + p, sz)],
                    sem,
                ))
            for c in copies:
                if do_wait:
                    c.wait()
                else:
                    c.start()

        lax.fori_loop(0, n64, lambda j, c: (pieces(j, 64, 0, False), c)[1], 0)
        lax.fori_loop(0, ntail,
                      lambda j, c: (pieces(j, 8, tbase, False), c)[1], 0)
        lax.fori_loop(0, n64, lambda j, c: (pieces(j, 64, 0, True), c)[1], 0)
        lax.fori_loop(0, ntail,
                      lambda j, c: (pieces(j, 8, tbase, True), c)[1], 0)


_sc_compact_call = functools.partial(
    pl.kernel,
    mesh=plsc.VectorSubcoreMesh(core_axis_name="c", subcore_axis_name="s",
                                num_cores=1),
    out_type=[
        jax.ShapeDtypeStruct((_B * 8 * _NPAD,), jnp.float32),
        jax.ShapeDtypeStruct((_B * _NPAD * 16,), jnp.float32),
        jax.ShapeDtypeStruct((16,), jnp.int32),
    ],
    scratch_types=[
        pltpu.VMEM((11 * _TR,), jnp.float32),
        pltpu.VMEM((8 * (_TR + 16),), jnp.float32),
        pltpu.VMEM(((_TR + 16) * 16,), jnp.float32),
        pltpu.VMEM_SHARED((256,), jnp.int32),
        pltpu.VMEM((16,), jnp.int32),
        pltpu.VMEM((256,), jnp.int32),
        pltpu.VMEM((16,), jnp.int32),
        pltpu.SemaphoreType.DMA,
    ],
    compiler_params=pltpu.CompilerParams(needs_layout_passes=False),
)(_sc_compact)


def _nms_kernel(craw_ref, ctab_ref, span_ref, out_ref, d_ref):
    qio = (
        jax.lax.broadcasted_iota(jnp.int32, (_NCH, 8, 256), 0) * _CHUNK
        + jax.lax.broadcasted_iota(jnp.int32, (_NCH, 8, 256), 1) * 256
        + jax.lax.broadcasted_iota(jnp.int32, (_NCH, 8, 256), 2)
    )
    qio_c = qio[0:1]

    maccs = []
    spans = []
    for img in range(_B):
        cx = craw_ref[img, 0]
        cy = craw_ref[img, 1]
        w = craw_ref[img, 2]
        h = craw_ref[img, 3]
        obj = craw_ref[img, 4]
        hw = w * 0.5
        hh = h * 0.5
        x1 = cx - hw
        y1 = cy - hh
        x2 = cx + hw
        y2 = cy + hh
        c0 = craw_ref[img, 5] * obj
        c1 = craw_ref[img, 6] * obj
        c2 = craw_ref[img, 7] * obj
        conf = jnp.maximum(jnp.maximum(c0, c1), c2)
        span = span_ref[img]
        valid = (obj > _CONF_THRES) & (conf > _CONF_THRES) & (qio < span)
        score = jnp.where(valid, conf, _NEG_INF)
        d_ref[img, 0] = x1
        d_ref[img, 1] = y1
        d_ref[img, 2] = x2
        d_ref[img, 3] = y2
        d_ref[img, 4] = (x2 - x1) * (y2 - y1)
        d_ref[img, 5] = score
        maccs.append(jnp.max(score, axis=0))
        spans.append(span)

    ncmax = (jnp.maximum(
        jnp.maximum(spans[0], spans[1]),
        jnp.maximum(spans[2], spans[3])) + (_CHUNK - 1)) // _CHUNK

    def body(i, maccs):
        ms = [jnp.max(maccs[img]) for img in range(_B)]
        idxs = []
        for img in range(_B):
            cand = jnp.where(d_ref[img, 5] == ms[img], qio, _BIG)
            idxs.append(jnp.min(cand))

        rows = []
        for img in range(_B):
            idx = idxs[img]
            t = idx // 64
            rem = idx - t * 64
            su = rem // 8
            u = rem - su * 8
            chunk = ctab_ref[img, pl.ds(t, 1)]
            chunk = pltpu.roll(chunk, (8 - su) % 8, axis=1)
            rows.append(pltpu.roll(chunk, (128 - u * 16) % 128, axis=2))

        binfo = []
        for img in range(_B):
            chunk = rows[img]
            bcx = chunk[0, 0, 0]
            bcy = chunk[0, 0, 1]
            bw = chunk[0, 0, 2]
            bh = chunk[0, 0, 3]
            bobj = chunk[0, 0, 4]
            bc0 = chunk[0, 0, 5] * bobj
            bc1 = chunk[0, 0, 6] * bobj
            bc2 = chunk[0, 0, 7] * bobj
            bl0 = chunk[0, 0, 8]
            bl1 = chunk[0, 0, 9]
            bl2 = chunk[0, 0, 10]
            bhw = bw * 0.5
            bhh = bh * 0.5
            bx1 = bcx - bhw
            by1 = bcy - bhh
            bx2 = bcx + bhw
            by2 = bcy + bhh
            barea = (bx2 - bx1) * (by2 - by1)
            bm01 = jnp.maximum(bc0, bc1)
            bconf = jnp.maximum(bm01, bc2)
            bj = jnp.where(bc1 > bc0, 1.0, 0.0)
            bj = jnp.where(bc2 > bm01, 2.0, bj)
            binfo.append((bx1, by1, bx2, by2, barea, bconf, bj, bl0, bl1, bl2))

        def supp_body(j2, accs):
            new = list(accs)
            for half in range(2):
                jc = j2 * 2 + half
                for img in range(_B):
                    bx1, by1, bx2, by2, barea = binfo[img][:5]
                    sc = d_ref[img, 5, pl.ds(jc, 1)][0]
                    ix1 = jnp.maximum(d_ref[img, 0, pl.ds(jc, 1)][0], bx1)
                    iy1 = jnp.maximum(d_ref[img, 1, pl.ds(jc, 1)][0], by1)
                    ix2 = jnp.minimum(d_ref[img, 2, pl.ds(jc, 1)][0], bx2)
                    iy2 = jnp.minimum(d_ref[img, 3, pl.ds(jc, 1)][0], by2)
                    inter = (jnp.maximum(ix2 - ix1, 0.0)
                             * jnp.maximum(iy2 - iy1, 0.0))
                    iou = inter / (barea + d_ref[img, 4, pl.ds(jc, 1)][0]
                                   - inter + 1e-9)
                    qc = qio_c[0] + jc * _CHUNK
                    supp = (iou > _IOU_THRES) | (qc == idxs[img])
                    snew = jnp.where(supp, _NEG_INF, sc)
                    d_ref[img, 5, pl.ds(jc, 1)] = snew[None]
                    new[img] = jnp.maximum(new[img], snew)
            return tuple(new)

        init = tuple(jnp.full((8, 256), _NEG_INF, jnp.float32)
                     for _ in range(_B))
        maccs = lax.fori_loop(0, (ncmax + 1) // 2, supp_body, init)

        for img in range(_B):
            bx1, by1, bx2, by2, barea, bconf, bj, bl0, bl1, bl2 = binfo[img]
            mok = jnp.where(ms[img] > _NEG_INF, 1.0, 0.0)
            zero = jnp.float32(0.0)
            vals = jnp.stack(
                [bx1, by1, bx2, by2, bconf, bj, bl0, bl1, bl2,
                 zero, zero, zero, zero, zero, zero, zero]
            ) * mok
            out_ref[img, pl.ds(i, 1)] = vals.reshape(1, 1, 16)
        return maccs

    lax.fori_loop(0, _MAX_DET, body, tuple(maccs))


@jax.jit
def kernel(prediction, logits):
    allc = jnp.concatenate([prediction, logits], axis=-1)
    chans = jnp.pad(allc.transpose(0, 2, 1), ((0, 0), (0, 0), (0, _NPAD - _N)))

    craw, ctab, span = _sc_compact_call(chans.reshape(-1))

    out = pl.pallas_call(
        _nms_kernel,
        in_specs=[
            pl.BlockSpec(memory_space=pltpu.VMEM),
            pl.BlockSpec(memory_space=pltpu.VMEM),
            pl.BlockSpec(memory_space=pltpu.SMEM),
        ],
        out_specs=pl.BlockSpec(memory_space=pltpu.VMEM),
        scratch_shapes=[pltpu.VMEM((_B, 6, _NCH, 8, 256), jnp.float32)],
        out_shape=jax.ShapeDtypeStruct((_B, _MAX_DET, 1, 16), jnp.float32),
    )(
        craw.reshape(_B, 8, _NCH, 8, 256),
        ctab.reshape(_B, _NPAD // 64, 8, 128),
        span,
    )

    outt = out.reshape(_B, _MAX_DET, 16)
    return outt[:, :, 0:6], outt[:, :, 6:9]
